# Initial kernel scaffold; baseline (speedup 1.0000x reference)
import jax
import jax.numpy as jnp
from jax.experimental import pallas as pl


def kernel(x, edge_index, idx, W_red, b_red, W_conv, W_ih, W_hh, b_ih, b_hh, W_lin, b_lin):
    raise NotImplementedError("write your pallas kernel here")


# speedup vs baseline: 8.7077x; 8.7077x over previous
import functools

import jax
import jax.numpy as jnp
from jax import lax
from jax.experimental import pallas as pl
from jax.experimental.pallas import tpu as pltpu
from jax.experimental.pallas import tpu_sc as plsc

NC = 2
NS = 16
W = 128
TR = 128


def _init_body(x_ref, w_ref, b_ref, o_ref):
    o_ref[...] = (jnp.dot(x_ref[...], w_ref[...],
                          preferred_element_type=jnp.float32) + b_ref[...])


def _init_h(x, w_red_t, b_red, n, npad, blk):
    return pl.pallas_call(
        _init_body,
        grid=(n // blk,),
        in_specs=[
            pl.BlockSpec((blk, 128), lambda i: (i, 0)),
            pl.BlockSpec((128, 128), lambda i: (0, 0)),
            pl.BlockSpec((1, 128), lambda i: (0, 0)),
        ],
        out_specs=pl.BlockSpec((blk, 128), lambda i: (i, 0)),
        out_shape=jax.ShapeDtypeStruct((npad, 128), jnp.float32),
    )(x, w_red_t, b_red)


def _fold_body(wc_ref, wt_ref, u_ref):
    u_ref[0] = jnp.dot(wc_ref[0], wt_ref[...],
                       preferred_element_type=jnp.float32)


def _fold_u(w_conv, w_ih_t, steps):
    return pl.pallas_call(
        _fold_body,
        grid=(steps,),
        in_specs=[
            pl.BlockSpec((1, 128, 128), lambda i: (i, 0, 0)),
            pl.BlockSpec((128, 384), lambda i: (0, 0)),
        ],
        out_specs=pl.BlockSpec((1, 128, 384), lambda i: (i, 0, 0)),
        out_shape=jax.ShapeDtypeStruct((steps, 128, 384), jnp.float32),
    )(w_conv, w_ih_t)


def _gru_body(h_ref, p_ref, u_ref, whh_ref, bih_ref, bhh_ref, *rest):
    if len(rest) == 1:
        o_ref, extra = rest[0], ()
    else:
        wl_ref, bl_ref, o_ref, y_ref = rest
        extra = (wl_ref, bl_ref, y_ref)
    d = 128
    h = h_ref[...]
    gi = (jnp.dot(p_ref[...], u_ref[...], preferred_element_type=jnp.float32)
          + bih_ref[...])
    gh = (jnp.dot(h, whh_ref[...], preferred_element_type=jnp.float32)
          + bhh_ref[...])
    r = jax.nn.sigmoid(gi[:, :d] + gh[:, :d])
    z = jax.nn.sigmoid(gi[:, d:2 * d] + gh[:, d:2 * d])
    n = jnp.tanh(gi[:, 2 * d:] + r * gh[:, 2 * d:])
    hn = (1.0 - z) * n + z * h
    o_ref[...] = hn
    if extra:
        wl_ref, bl_ref, y_ref = extra
        y = jnp.dot(hn, wl_ref[...], preferred_element_type=jnp.float32)
        y_ref[...] = jax.nn.sigmoid(y + bl_ref[...])


def _gru_step(h, p, u_i, whh_t, b_ih, b_hh, n, npad, blk, last=None):
    grid = (n // blk,)
    in_specs = [
        pl.BlockSpec((blk, 128), lambda i: (i, 0)),
        pl.BlockSpec((blk, 128), lambda i: (i, 0)),
        pl.BlockSpec((128, 384), lambda i: (0, 0)),
        pl.BlockSpec((128, 384), lambda i: (0, 0)),
        pl.BlockSpec((1, 384), lambda i: (0, 0)),
        pl.BlockSpec((1, 384), lambda i: (0, 0)),
    ]
    out_specs = pl.BlockSpec((blk, 128), lambda i: (i, 0))
    out_shape = jax.ShapeDtypeStruct((npad, 128), jnp.float32)
    args = [h, p, u_i, whh_t, b_ih, b_hh]
    if last is None:
        return pl.pallas_call(
            _gru_body, grid=grid, in_specs=in_specs,
            out_specs=out_specs, out_shape=out_shape)(*args)
    wl, bl = last
    in_specs += [
        pl.BlockSpec((128, 1), lambda i: (0, 0)),
        pl.BlockSpec((1, 1), lambda i: (0, 0)),
    ]
    return pl.pallas_call(
        _gru_body, grid=grid, in_specs=in_specs,
        out_specs=[out_specs, pl.BlockSpec((blk, 1), lambda i: (i, 0))],
        out_shape=[out_shape, jax.ShapeDtypeStruct((n, 1), jnp.float32)],
    )(*(args + [wl, bl]))


NBUF = 4
CH = 8


def _make_segsum(npad, nchunk):
    half = npad // NC
    rows_pt = half // NS
    nagg = half + TR
    mesh = plsc.VectorSubcoreMesh(core_axis_name="c", subcore_axis_name="s",
                                  num_cores=NC, num_subcores=NS)

    @functools.partial(
        pl.kernel, mesh=mesh,
        out_type=jax.ShapeDtypeStruct((npad, 128), jnp.float32),
        scratch_types=[
            pltpu.VMEM((2, CH, W), jnp.int32),
            pltpu.VMEM((2, CH, W), jnp.int32),
            pltpu.VMEM((NBUF, W, 128), jnp.float32),
            pltpu.VMEM((16,), jnp.int32),
            pltpu.VMEM_SHARED((nagg, 128), jnp.float32),
            pltpu.SemaphoreType.DMA((NBUF,)),
            pltpu.SemaphoreType.DMA((NBUF,)),
            pltpu.SemaphoreType.DMA((2,)),
        ],
    )
    def segsum(h, srcw, dstw, zeros, cnts, p, src_v, dst_v, rows_v, cnt_v,
               agg_sh, gsem, ssem, isem):
        c = lax.axis_index("c")
        s = lax.axis_index("s")
        r0 = s * rows_pt
        pltpu.sync_copy(zeros.at[pl.ds(r0, rows_pt)],
                        agg_sh.at[pl.ds(r0, rows_pt)])
        pltpu.sync_copy(cnts.at[c], cnt_v)
        nch = cnt_v[...][0]

        def _prefetch(k):
            st = k % 2
            pltpu.async_copy(srcw.at[c, s, pl.ds(k * CH, CH)], src_v.at[st],
                             isem.at[st])
            pltpu.async_copy(dstw.at[c, s, pl.ds(k * CH, CH)], dst_v.at[st],
                             isem.at[st])

        def _wait_idx(k):
            st = k % 2
            pltpu.make_async_copy(srcw.at[c, s, pl.ds(k * CH, CH)],
                                  src_v.at[st], isem.at[st]).wait()
            pltpu.make_async_copy(dstw.at[c, s, pl.ds(k * CH, CH)],
                                  dst_v.at[st], isem.at[st]).wait()

        def _wait_scat(b):
            pltpu.make_async_copy(rows_v.at[b], agg_sh.at[dst_v.at[0, 0]],
                                  ssem.at[b]).wait()

        _prefetch(0)
        plsc.subcore_barrier()

        for k in range(nchunk):
            st = k % 2

            @pl.when(k < nch)
            def _chunk():
                if k > 0:
                    for b in range(NBUF):
                        _wait_scat(b)
                if k + 1 < nchunk:
                    @pl.when(k + 1 < nch)
                    def _():
                        _prefetch(k + 1)
                _wait_idx(k)

                def body(g, carry):
                    w0 = g * NBUF
                    for b in range(NBUF):
                        @pl.when(g > 0)
                        def _():
                            _wait_scat(b)
                        pltpu.async_copy(h.at[src_v.at[st, w0 + b]],
                                         rows_v.at[b], gsem.at[b])
                    for b in range(NBUF):
                        pltpu.make_async_copy(h.at[src_v.at[st, w0 + b]],
                                              rows_v.at[b], gsem.at[b]).wait()
                        pltpu.async_copy(rows_v.at[b],
                                         agg_sh.at[dst_v.at[st, w0 + b]],
                                         ssem.at[b], add=True)
                    return carry

                lax.fori_loop(0, CH // NBUF, body, 0)

        for b in range(NBUF):
            _wait_scat(b)
        plsc.subcore_barrier()
        pltpu.sync_copy(agg_sh.at[pl.ds(r0, rows_pt)],
                        p.at[pl.ds(c * half + r0, rows_pt)])

    return segsum


def _make_select(n, nsel_pad):
    per_w = nsel_pad // (NC * NS)
    mesh = plsc.VectorSubcoreMesh(core_axis_name="c", subcore_axis_name="s",
                                  num_cores=NC, num_subcores=NS)

    @functools.partial(
        pl.kernel, mesh=mesh,
        out_type=jax.ShapeDtypeStruct((nsel_pad,), jnp.float32),
        scratch_types=[
            pltpu.VMEM((per_w,), jnp.int32),
            pltpu.VMEM((per_w,), jnp.float32),
        ],
    )
    def select(y, idxp, out, idx_v, val_v):
        c = lax.axis_index("c")
        s = lax.axis_index("s")
        wid = s * NC + c
        base = wid * per_w
        pltpu.sync_copy(idxp.at[pl.ds(base, per_w)], idx_v)
        pltpu.sync_copy(y.at[idx_v], val_v)
        pltpu.sync_copy(val_v, out.at[pl.ds(base, per_w)])

    return select


def kernel(x, edge_index, idx, W_red, b_red, W_conv, W_ih, W_hh, b_ih, b_hh,
           W_lin, b_lin):
    n, d = x.shape
    steps = W_conv.shape[0]
    e = edge_index.shape[1]
    nsel = idx.shape[0]
    blk = 2000

    half = NS * 8 * -(-n // (NC * NS * 8))
    npad = NC * half
    nchunk = -(-e // (NS * W * CH))
    nwin = nchunk * CH
    epad = NS * W * nwin

    src = edge_index[0].astype(jnp.int32)
    dst = edge_index[1].astype(jnp.int32)
    inh0 = dst < half
    cum0 = jnp.cumsum(inh0.astype(jnp.int32))
    ei = jnp.arange(e, dtype=jnp.int32)
    pos = jnp.where(inh0, cum0 - 1, epad + ei - cum0)
    dstloc = jnp.where(inh0, dst, dst - half)
    srcp = jnp.zeros((NC * epad,), jnp.int32).at[pos].add(
        src + 1, unique_indices=True)
    dstp = jnp.zeros((NC * epad,), jnp.int32).at[pos].add(
        dstloc + 1, unique_indices=True)
    ar2 = jnp.arange(NC * epad, dtype=jnp.int32)
    src_f = jnp.where(srcp == 0, (ar2 * 37) % n, srcp - 1)
    dst_f = jnp.where(dstp == 0, half + ar2 % TR, dstp - 1)
    srcw = src_f.reshape(NC, nwin, NS, W).transpose(0, 2, 1, 3)
    dstw = dst_f.reshape(NC, nwin, NS, W).transpose(0, 2, 1, 3)
    k0 = cum0[-1]
    kc = jnp.stack([k0, e - k0])
    wpt = (-(-kc // W) + NS - 1) // NS
    active = jnp.clip((wpt + CH - 1) // CH, 1, nchunk).astype(jnp.int32)
    cnts = jnp.broadcast_to(active[:, None], (NC, 16))
    zeros = jnp.zeros((half, 128), jnp.float32)

    nsel_pad = NC * NS * 64 * -(-nsel // (NC * NS * 64))
    idxp = jnp.concatenate(
        [idx, jnp.zeros((nsel_pad - nsel,), idx.dtype)]).astype(jnp.int32)

    w_red_t = W_red.T
    w_ih_t = W_ih.T
    whh_t = W_hh.T
    b_ih2 = b_ih.reshape(1, 384)
    b_hh2 = b_hh.reshape(1, 384)
    wl = W_lin.T
    bl2 = b_lin.reshape(1, 1)
    b_red2 = b_red.reshape(1, 128)

    u = _fold_u(W_conv, w_ih_t, steps)
    h = _init_h(x, w_red_t, b_red2, n, npad, blk)

    segsum = _make_segsum(npad, nchunk)
    for i in range(steps):
        p = segsum(h, srcw, dstw, zeros, cnts)
        if i < steps - 1:
            h = _gru_step(h, p, u[i], whh_t, b_ih2, b_hh2, n, npad, blk)
        else:
            h, y = _gru_step(h, p, u[i], whh_t, b_ih2, b_hh2, n, npad,
                             blk, last=(wl, bl2))

    sel = _make_select(n, nsel_pad)(y.reshape(n), idxp)
    return sel[:nsel].reshape(nsel, 1)
